# baseline (speedup 1.0000x reference)
import jax
import jax.numpy as jnp
from jax.experimental import pallas as pl
from jax.experimental.pallas import tpu as pltpu

_B, _C, _H, _W = 4, 96, 256, 256
_BC = 8
_GJ = _C // _BC
_N = _B * _GJ
_K = 24
_SCALE = 127.0


def _in_map(p, i, j):
    n = i * _GJ + j
    cached = (p == 1) & (n < _K)
    return (jnp.where(cached, _B - 1, i), jnp.where(cached, _GJ - 1, j), 0, 0)


def _out_map(p, i, j):
    return (jnp.where(p == 0, 0, i), jnp.where(p == 0, 0, j), 0, 0)


def _body(x_ref, o_ref, min_ref, max_ref, cache, acc_min, acc_max, sca):
    p = pl.program_id(0)
    i = pl.program_id(1)
    j = pl.program_id(2)
    n = i * _GJ + j

    @pl.when(p == 0)
    def _phase0():
        @pl.when(n == 0)
        def _init():
            acc_min[...] = jnp.full_like(acc_min, jnp.inf)
            acc_max[...] = jnp.full_like(acc_max, -jnp.inf)

        x = x_ref[...]
        xv = x.reshape(_BC * _H, _W)
        acc_min[...] = jnp.minimum(acc_min[...], jnp.min(xv, axis=0, keepdims=True))
        acc_max[...] = jnp.maximum(acc_max[...], jnp.max(xv, axis=0, keepdims=True))

        @pl.when(n < _K)
        def _stash():
            cache[pl.ds(n, 1)] = x

        @pl.when(n == _N - 1)
        def _finish():
            x_min = jnp.min(acc_min[...])
            x_max = jnp.max(acc_max[...])
            sca[0] = x_min
            sca[1] = x_max
            min_ref[0, 0] = x_min
            max_ref[0, 0] = x_max

    @pl.when(p == 1)
    def _phase1():
        x_min = sca[0]
        x_max = sca[1]

        def quant(x):
            xn = (x - x_min) / (x_max - x_min) * 2.0 - 1.0
            return jnp.round(xn * _SCALE) / _SCALE

        @pl.when(n < _K)
        def _from_cache():
            o_ref[...] = quant(cache[pl.ds(n, 1)])

        @pl.when(n >= _K)
        def _from_hbm():
            o_ref[...] = quant(x_ref[...])


def kernel(x_ll):
    q, x_min, x_max = pl.pallas_call(
        _body,
        grid=(2, _B, _GJ),
        in_specs=[pl.BlockSpec((1, _BC, _H, _W), _in_map)],
        out_specs=[
            pl.BlockSpec((1, _BC, _H, _W), _out_map),
            pl.BlockSpec(memory_space=pltpu.SMEM),
            pl.BlockSpec(memory_space=pltpu.SMEM),
        ],
        out_shape=[
            jax.ShapeDtypeStruct((_B, _C, _H, _W), jnp.float32),
            jax.ShapeDtypeStruct((1, 1), jnp.float32),
            jax.ShapeDtypeStruct((1, 1), jnp.float32),
        ],
        scratch_shapes=[
            pltpu.VMEM((_K, _BC, _H, _W), jnp.float32),
            pltpu.VMEM((1, _W), jnp.float32),
            pltpu.VMEM((1, _W), jnp.float32),
            pltpu.SMEM((2,), jnp.float32),
        ],
    )(x_ll)

    return (q, x_min.reshape(()), x_max.reshape(()))
